# baseline (speedup 1.0000x reference)
import functools

import jax
import jax.numpy as jnp
from jax import lax
from jax.experimental import pallas as pl
from jax.experimental.pallas import tpu as pltpu
from jax.experimental.pallas import tpu_sc as plsc

B = 16384
F = 26
V = 100000
D = 32

NC = 2
NS = 16
NW = NC * NS
L = 16

TPW = (F * D) // NW
HB = 8192


def _body(xt_hbm, tbl_hbm, out_hbm, slc_v, idx_v, val_v):
    cid = lax.axis_index("c")
    sid = lax.axis_index("s")
    wid = sid * NC + cid
    t0 = wid * TPW

    def task_body(ti, carry):
        t = t0 + ti
        f = t >> 5
        d = t & 31
        pltpu.sync_copy(tbl_hbm.at[f, d], slc_v)

        def half_body(h, carry2):
            b0 = h * HB
            pltpu.sync_copy(xt_hbm.at[f, pl.ds(b0, HB)], idx_v)

            @plsc.parallel_loop(0, HB, step=L, unroll=8)
            def gather_body(i):
                vec = idx_v[pl.ds(i, L)]
                val_v[pl.ds(i, L)] = plsc.load_gather(slc_v, [vec])
            pltpu.sync_copy(val_v, out_hbm.at[t, pl.ds(b0, HB)])
            return carry2

        lax.fori_loop(0, B // HB, half_body, 0)
        return carry

    lax.fori_loop(0, TPW, task_body, 0)


@jax.jit
def _gather(xt, tbl):
    k = functools.partial(
        pl.kernel,
        out_type=jax.ShapeDtypeStruct((F * D, B), jnp.float32),
        mesh=plsc.VectorSubcoreMesh(core_axis_name="c", subcore_axis_name="s"),
        scratch_types=[
            pltpu.VMEM((V,), jnp.float32),
            pltpu.VMEM((HB,), jnp.int32),
            pltpu.VMEM((HB,), jnp.float32),
        ],
        compiler_params=pltpu.CompilerParams(
            needs_layout_passes=False,
            disable_bounds_checks=True,
        ),
    )(_body)
    return k(xt, tbl)


def kernel(x, tables):
    xt = x.T
    tbl = tables.transpose(0, 2, 1)
    out_t = _gather(xt, tbl)
    return out_t.T


# speedup vs baseline: 1.3089x; 1.3089x over previous
import functools

import jax
import jax.numpy as jnp
from jax import lax
from jax.experimental import pallas as pl
from jax.experimental.pallas import tpu as pltpu
from jax.experimental.pallas import tpu_sc as plsc

B = 16384
F = 26
V = 100000
D = 32

NC = 2
NS = 16
NW = NC * NS
L = 16

TPW = (F * D) // NW
HBV = 4096
NCK = B // HBV


def _body(xt_hbm, tbl_hbm, out_hbm, slc_v, idx_v, val0, val1, sem_s, sw0, sw1):
    cid = lax.axis_index("c")
    sid = lax.axis_index("s")
    wid = sid * NC + cid
    t0 = wid * TPW

    vals = (val0, val1)
    sems_w = (sw0, sw1)
    wdesc = [None, None]

    sdesc = None
    for ti in range(TPW):
        t = t0 + ti
        f = t >> 5
        d = t & 31
        if ti == 0:
            sdesc = pltpu.async_copy(tbl_hbm.at[f, d], slc_v, sem_s)
        sdesc.wait()

        if ti == 0:
            pltpu.sync_copy(xt_hbm.at[f], idx_v)
        else:
            @pl.when(d == 0)
            def _load_idx():
                pltpu.sync_copy(xt_hbm.at[f], idx_v)

        for c in range(NCK):
            cb = (ti * NCK + c) % 2
            if wdesc[cb] is not None:
                wdesc[cb].wait()
            vbuf = vals[cb]

            @plsc.parallel_loop(0, HBV, step=L, unroll=8)
            def gather_body(i):
                vec = idx_v[pl.ds(c * HBV + i, L)]
                vbuf[pl.ds(i, L)] = plsc.load_gather(slc_v, [vec])

            if c == NCK - 1 and ti + 1 < TPW:
                tn = t + 1
                sdesc = pltpu.async_copy(
                    tbl_hbm.at[tn >> 5, tn & 31], slc_v, sem_s
                )
            wdesc[cb] = pltpu.async_copy(
                vbuf, out_hbm.at[t, pl.ds(c * HBV, HBV)], sems_w[cb]
            )

    wdesc[0].wait()
    wdesc[1].wait()


@jax.jit
def _gather(xt, tbl):
    k = functools.partial(
        pl.kernel,
        out_type=jax.ShapeDtypeStruct((F * D, B), jnp.float32),
        mesh=plsc.VectorSubcoreMesh(core_axis_name="c", subcore_axis_name="s"),
        scratch_types=[
            pltpu.VMEM((V,), jnp.float32),
            pltpu.VMEM((B,), jnp.int32),
            pltpu.VMEM((HBV,), jnp.float32),
            pltpu.VMEM((HBV,), jnp.float32),
            pltpu.SemaphoreType.DMA,
            pltpu.SemaphoreType.DMA,
            pltpu.SemaphoreType.DMA,
        ],
        compiler_params=pltpu.CompilerParams(
            needs_layout_passes=False,
            disable_bounds_checks=True,
        ),
    )(_body)
    return k(xt, tbl)


def kernel(x, tables):
    xt = x.T
    tbl = tables.transpose(0, 2, 1)
    out_t = _gather(xt, tbl)
    return out_t.T


# speedup vs baseline: 1.3097x; 1.0006x over previous
import functools

import jax
import jax.numpy as jnp
from jax import lax
from jax.experimental import pallas as pl
from jax.experimental.pallas import tpu as pltpu
from jax.experimental.pallas import tpu_sc as plsc

B = 16384
F = 26
V = 100000
D = 32

NC = 2
NS = 16
NW = NC * NS
L = 16

TPW = (F * D) // NW
HBV = 4096
NCK = B // HBV


def _fire_slice(tbl_hbm, f, d, slc_v, sem_s):
    return [pltpu.async_copy(tbl_hbm.at[f, d], slc_v, sem_s)]


def _body(xt_hbm, tbl_hbm, out_hbm, slc_v, idx_v, val0, val1, sem_s, sw0, sw1):
    cid = lax.axis_index("c")
    sid = lax.axis_index("s")
    wid = sid * NC + cid
    t0 = wid * TPW

    vals = (val0, val1)
    sems_w = (sw0, sw1)
    wdesc = [None, None]

    sdesc = None
    for ti in range(TPW):
        t = t0 + ti
        f = t >> 5
        d = t & 31
        if ti == 0:
            sdesc = _fire_slice(tbl_hbm, f, d, slc_v, sem_s)
        for sd in sdesc:
            sd.wait()

        if ti == 0:
            pltpu.sync_copy(xt_hbm.at[f], idx_v)
        else:
            @pl.when(d == 0)
            def _load_idx():
                pltpu.sync_copy(xt_hbm.at[f], idx_v)

        for c in range(NCK):
            cb = (ti * NCK + c) % 2
            if wdesc[cb] is not None:
                wdesc[cb].wait()
            vbuf = vals[cb]

            @plsc.parallel_loop(0, HBV, step=L, unroll=8)
            def gather_body(i):
                vec = idx_v[pl.ds(c * HBV + i, L)]
                vbuf[pl.ds(i, L)] = plsc.load_gather(slc_v, [vec])

            if c == NCK - 1 and ti + 1 < TPW:
                tn = t + 1
                sdesc = _fire_slice(tbl_hbm, tn >> 5, tn & 31, slc_v, sem_s)
            wdesc[cb] = pltpu.async_copy(
                vbuf, out_hbm.at[t, pl.ds(c * HBV, HBV)], sems_w[cb]
            )

    wdesc[0].wait()
    wdesc[1].wait()


@jax.jit
def _gather(xt, tbl):
    k = functools.partial(
        pl.kernel,
        out_type=jax.ShapeDtypeStruct((F * D, B), jnp.float32),
        mesh=plsc.VectorSubcoreMesh(core_axis_name="c", subcore_axis_name="s"),
        scratch_types=[
            pltpu.VMEM((V,), jnp.float32),
            pltpu.VMEM((B,), jnp.int32),
            pltpu.VMEM((HBV,), jnp.float32),
            pltpu.VMEM((HBV,), jnp.float32),
            pltpu.SemaphoreType.DMA,
            pltpu.SemaphoreType.DMA,
            pltpu.SemaphoreType.DMA,
        ],
        compiler_params=pltpu.CompilerParams(
            needs_layout_passes=False,
            disable_bounds_checks=True,
        ),
    )(_body)
    return k(xt, tbl)


def kernel(x, tables):
    xt = x.T
    tbl = tables.transpose(0, 2, 1)
    out_t = _gather(xt, tbl)
    return out_t.T
